# baseline (speedup 1.0000x reference)
import functools

import jax
import jax.numpy as jnp
from jax.experimental import pallas as pl
from jax.experimental.pallas import tpu as pltpu


def _edge_mlp_body(leaf_ref, w1_ref, b1_ref, w2_ref, b2_ref, out_ref):
    x = leaf_ref[...]
    h = jnp.dot(x, w1_ref[...], preferred_element_type=jnp.float32) + b1_ref[...]
    h = jnp.maximum(h, 0.0)
    out_ref[...] = (
        jnp.dot(h, w2_ref[...], preferred_element_type=jnp.float32) + b2_ref[...]
    )


def _edge_mlp(leaf, W1, b1, W2, b2, block_e=2048):
    E, F = leaf.shape
    H = W2.shape[1]
    grid = (pl.cdiv(E, block_e),)
    return pl.pallas_call(
        _edge_mlp_body,
        grid=grid,
        in_specs=[
            pl.BlockSpec((block_e, F), lambda i: (i, 0)),
            pl.BlockSpec((F, H), lambda i: (0, 0)),
            pl.BlockSpec((1, H), lambda i: (0, 0)),
            pl.BlockSpec((H, H), lambda i: (0, 0)),
            pl.BlockSpec((1, H), lambda i: (0, 0)),
        ],
        out_specs=pl.BlockSpec((block_e, H), lambda i: (i, 0)),
        out_shape=jax.ShapeDtypeStruct((E, H), jnp.float32),
        compiler_params=pltpu.CompilerParams(
            dimension_semantics=("parallel",),
        ),
    )(leaf, W1, b1.reshape(1, H), W2, b2.reshape(1, H))


def _node_body(center_ref, agg_ref, gcn_ref, w3_ref, b3_ref, w4_ref, b4_ref,
               out_ref):
    c = center_ref[...]
    a = agg_ref[...]
    H = c.shape[1]
    w3c = w3_ref[0:H, :]
    w3a = w3_ref[H:2 * H, :]
    h = (
        jnp.dot(c, w3c, preferred_element_type=jnp.float32)
        + jnp.dot(a, w3a, preferred_element_type=jnp.float32)
        + b3_ref[...]
    )
    h = jnp.maximum(h, 0.0)
    lang = jnp.dot(h, w4_ref[...], preferred_element_type=jnp.float32) + b4_ref[...]
    g = gcn_ref[...]
    num = jnp.sum(g * lang, axis=1)
    ng = jnp.maximum(jnp.sqrt(jnp.sum(g * g, axis=1)), 1e-8)
    nl = jnp.maximum(jnp.sqrt(jnp.sum(lang * lang, axis=1)), 1e-8)
    out_ref[...] = num / (ng * nl)


def _node_mlp_cosine(center, agg, gcn, W3, b3, W4, b4, block_n=2048):
    N, H = center.shape
    grid = (pl.cdiv(N, block_n),)
    return pl.pallas_call(
        _node_body,
        grid=grid,
        in_specs=[
            pl.BlockSpec((block_n, H), lambda i: (i, 0)),
            pl.BlockSpec((block_n, H), lambda i: (i, 0)),
            pl.BlockSpec((block_n, H), lambda i: (i, 0)),
            pl.BlockSpec((2 * H, H), lambda i: (0, 0)),
            pl.BlockSpec((1, H), lambda i: (0, 0)),
            pl.BlockSpec((H, H), lambda i: (0, 0)),
            pl.BlockSpec((1, H), lambda i: (0, 0)),
        ],
        out_specs=pl.BlockSpec((block_n,), lambda i: (i,)),
        out_shape=jax.ShapeDtypeStruct((N,), jnp.float32),
        compiler_params=pltpu.CompilerParams(
            dimension_semantics=("parallel",),
        ),
    )(center, agg, gcn, W3, b3.reshape(1, H), W4, b4.reshape(1, H))


def kernel(center_node_attr, leaf_node_all, node_idx, gcnfeats,
           W1, b1, W2, b2, W3, b3, W4, b4):
    n = center_node_attr.shape[0]
    msg = _edge_mlp(leaf_node_all, W1, b1, W2, b2)
    agg = jax.ops.segment_max(msg, node_idx, num_segments=n)
    agg = jnp.where(jnp.isneginf(agg), 0.0, agg)
    return _node_mlp_cosine(center_node_attr, agg, gcnfeats, W3, b3, W4, b4)
